# baseline (speedup 1.0000x reference)
import functools

import jax
import jax.numpy as jnp
from jax import lax
from jax.experimental import pallas as pl
from jax.experimental.pallas import tpu as pltpu
from jax.experimental.pallas import tpu_sc as plsc

NA = 32
NACT = 8
DPRE = 64
GPB = 16
R = NA * GPB

NODES_PER_W = 64


def _gate_sc_body(zt_hbm, w2_hbm, b2_hbm, out_hbm, zt_vm, w2_vm, b2_vm, w_vm):
    wid = lax.axis_index("s") * 2 + lax.axis_index("c")
    pltpu.sync_copy(zt_hbm.at[wid], zt_vm)
    pltpu.sync_copy(w2_hbm, w2_vm)
    pltpu.sync_copy(b2_hbm, b2_vm.at[pl.ds(0, 1)])

    zero16 = jnp.zeros((16,), jnp.float32)
    b2s = b2_vm[...][0]
    wrow = [w2_vm[pl.ds(h * 16, 16)] for h in range(4)]
    wa = [wrow[k // 16][k % 16] for k in range(NA)]
    wb = [wrow[2 + k // 16][k % 16] for k in range(NA)]
    u_vecs, v_vecs = [], []
    for h in range(NODES_PER_W // 16):
        u_vec, v_vec = zero16, zero16
        for k in range(NA):
            zt = zt_vm[k, pl.ds(h * 16, 16)]
            u_vec = u_vec + zt * wa[k]
            v_vec = v_vec + zt * wb[k]
        u_vecs.append(u_vec)
        v_vecs.append(v_vec)
    for g in range(NODES_PER_W // NA):
        u0, u1 = u_vecs[2 * g], u_vecs[2 * g + 1]
        for i in range(NA):
            arg = v_vecs[2 * g + i // 16][i % 16] + b2s
            r = (g * NA + i) * NA
            w_vm[pl.ds(r, 16)] = 1.0 / (1.0 + jnp.exp(-(u0 + arg)))
            w_vm[pl.ds(r + 16, 16)] = 1.0 / (1.0 + jnp.exp(-(u1 + arg)))

    pltpu.sync_copy(w_vm, out_hbm.at[pl.ds(wid * NODES_PER_W * NA,
                                           NODES_PER_W * NA)])


def _gate_sc(mypose_goalpose, W_attn_w, b_attn_w):
    n = mypose_goalpose.shape[0]
    mesh = plsc.VectorSubcoreMesh(core_axis_name="c", subcore_axis_name="s")
    k = functools.partial(
        pl.kernel, mesh=mesh,
        out_type=jax.ShapeDtypeStruct((n * NA,), jnp.float32),
        scratch_types=[
            pltpu.VMEM((NA, NODES_PER_W), jnp.float32),
            pltpu.VMEM((2 * NA,), jnp.float32),
            pltpu.VMEM((16,), jnp.float32),
            pltpu.VMEM((NODES_PER_W * NA,), jnp.float32),
        ],
    )(_gate_sc_body)
    zt3 = mypose_goalpose.T.reshape(NA, 32, NODES_PER_W).transpose(1, 0, 2)
    wflat = k(zt3, W_attn_w.reshape(-1), b_attn_w)
    return wflat.reshape(n, NA)


def _critic_kernel(obs_ref, z_ref, pol_ref, act_ref,
                   wfc_ref, bfc_ref, w1_ref, b1_ref, w2_ref, b2_ref,
                   wv_ref, bv_ref, x_ref):
    f32 = jnp.float32
    dot = functools.partial(jax.lax.dot_general,
                            preferred_element_type=f32)
    mm = lambda a, b: dot(a, b, (((1,), (0,)), ((), ())))
    mm_t = lambda a, b: dot(a, b, (((1,), (1,)), ((), ())))

    row_g = jax.lax.broadcasted_iota(jnp.int32, (R, R), 0) // NA
    col_g = jax.lax.broadcasted_iota(jnp.int32, (R, R), 1) // NA
    mask = (row_g == col_g).astype(f32)
    sel_c = jax.lax.broadcasted_iota(jnp.int32, (R, NA), 0) % NA
    sel_j = jax.lax.broadcasted_iota(jnp.int32, (R, NA), 1)
    T = (sel_c == sel_j).astype(f32)

    w1a = w1_ref[0:1, :DPRE]
    w1b = w1_ref[0:1, DPRE:]
    w2a = w2_ref[0:1, :NA]
    w2b = w2_ref[0:1, NA:]
    wv1 = wv_ref[0:1, :DPRE]
    wv2 = wv_ref[0:1, DPRE:DPRE + NACT]
    b1 = b1_ref[0]
    b2 = b2_ref[0]
    bv = bv_ref[0]

    obs = obs_ref[...]
    F = mm_t(obs, wfc_ref[...]) + bfc_ref[...]

    a_row = mm_t(w1a, F)
    c_col = mm_t(F, w1b)
    alpha = jax.nn.sigmoid(c_col + a_row + b1) * mask
    obs_proc = mm(alpha, F)
    s_col = mm_t(obs_proc, wv1)

    z = z_ref[...]
    u_row = mm_t(w2a, z)
    v_col = mm_t(z, w2b)
    wfull = jax.nn.sigmoid(v_col + u_row + b2) * mask

    pol = pol_ref[...]
    dp_col = mm_t(act_ref[...] - pol, wv2)
    pp_col = mm_t(pol, wv2)

    rhs_w = jnp.concatenate([T, dp_col * (1.0 / NA)], axis=1)
    ws = mm(wfull, rhs_w)
    w_out = ws[:, :NA]
    S_col = ws[:, NA:NA + 1]
    rhs_m = jnp.concatenate([T * s_col, T * dp_col, pp_col * (1.0 / NA)],
                            axis=1)
    rows = mm(mask, rhs_m)
    s_rows = rows[:, :NA]
    dp_rows = rows[:, NA:2 * NA]
    pm_col = rows[:, 2 * NA:2 * NA + 1]

    x_ref[...] = (S_col + pm_col + bv) + s_rows \
        - w_out * dp_rows * (1.0 / NA)


def kernel(obs, mypose_goalpose, policies, actions,
           W_fc, b_fc, W_attn_in, b_attn_in, W_attn_w, b_attn_w,
           W_val, b_val):
    n = obs.shape[0]
    grid = n // R

    row_spec = lambda w: pl.BlockSpec((R, w), lambda i: (i, 0))
    full = lambda a: pl.BlockSpec(a.shape, lambda *_: (0,) * a.ndim)

    x2d = pl.pallas_call(
        _critic_kernel,
        grid=(grid,),
        in_specs=[
            row_spec(obs.shape[1]),
            row_spec(mypose_goalpose.shape[1]),
            row_spec(NACT),
            row_spec(NACT),
            full(W_fc), full(b_fc),
            full(W_attn_in), full(b_attn_in),
            full(W_attn_w), full(b_attn_w),
            full(W_val), full(b_val),
        ],
        out_specs=row_spec(NA),
        out_shape=jax.ShapeDtypeStruct((n, NA), jnp.float32),
    )(obs, mypose_goalpose, policies, actions,
      W_fc, b_fc, W_attn_in, b_attn_in, W_attn_w, b_attn_w, W_val, b_val)

    w2d = _gate_sc(mypose_goalpose, W_attn_w, b_attn_w)

    return x2d[:, :, None], w2d[:, :, None]


# speedup vs baseline: 1.8928x; 1.8928x over previous
import functools

import jax
import jax.numpy as jnp
from jax.experimental import pallas as pl

NA = 32
NACT = 8
DPRE = 64
GPB = 16
R = NA * GPB


def _critic_kernel(obs_ref, z_ref, pol_ref, act_ref,
                   wfc_ref, bfc_ref, w1_ref, b1_ref, w2_ref, b2_ref,
                   wv_ref, bv_ref, x_ref, w_ref):
    f32 = jnp.float32
    dot = functools.partial(jax.lax.dot_general,
                            preferred_element_type=f32)
    mm = lambda a, b: dot(a, b, (((1,), (0,)), ((), ())))
    mm_t = lambda a, b: dot(a, b, (((1,), (1,)), ((), ())))

    row_g = jax.lax.broadcasted_iota(jnp.int32, (R, R), 0) // NA
    col_g = jax.lax.broadcasted_iota(jnp.int32, (R, R), 1) // NA
    mask = (row_g == col_g).astype(f32)
    sel_c = jax.lax.broadcasted_iota(jnp.int32, (R, NA), 0) % NA
    sel_j = jax.lax.broadcasted_iota(jnp.int32, (R, NA), 1)
    T = (sel_c == sel_j).astype(f32)

    w1a = w1_ref[0:1, :DPRE]
    w1b = w1_ref[0:1, DPRE:]
    w2a = w2_ref[0:1, :NA]
    w2b = w2_ref[0:1, NA:]
    wv1 = wv_ref[0:1, :DPRE]
    wv2 = wv_ref[0:1, DPRE:DPRE + NACT]
    b1 = b1_ref[0]
    b2 = b2_ref[0]
    bv = bv_ref[0]

    obs = obs_ref[...]
    F = mm_t(obs, wfc_ref[...]) + bfc_ref[...]

    a_row = mm_t(w1a, F)
    c_col = mm_t(F, w1b)
    alpha = jax.nn.sigmoid(c_col + a_row + b1) * mask
    obs_proc = mm(alpha, F)
    s_col = mm_t(obs_proc, wv1)

    z = z_ref[...]
    u_row = mm_t(w2a, z)
    v_col = mm_t(z, w2b)
    wfull = jax.nn.sigmoid(v_col + u_row + b2) * mask

    pol = pol_ref[...]
    dp_col = mm_t(act_ref[...] - pol, wv2)
    pp_col = mm_t(pol, wv2)

    rhs_w = jnp.concatenate([T, dp_col * (1.0 / NA)], axis=1)
    ws = mm(wfull, rhs_w)
    w_out = ws[:, :NA]
    S_col = ws[:, NA:NA + 1]
    rhs_m = jnp.concatenate([T * s_col, T * dp_col, pp_col * (1.0 / NA)],
                            axis=1)
    rows = mm(mask, rhs_m)
    s_rows = rows[:, :NA]
    dp_rows = rows[:, NA:2 * NA]
    pm_col = rows[:, 2 * NA:2 * NA + 1]

    w_ref[...] = w_out
    x_ref[...] = (S_col + pm_col + bv) + s_rows \
        - w_out * dp_rows * (1.0 / NA)


def kernel(obs, mypose_goalpose, policies, actions,
           W_fc, b_fc, W_attn_in, b_attn_in, W_attn_w, b_attn_w,
           W_val, b_val):
    n = obs.shape[0]
    grid = n // R

    row_spec = lambda w: pl.BlockSpec((R, w), lambda i: (i, 0))
    full = lambda a: pl.BlockSpec(a.shape, lambda *_: (0,) * a.ndim)

    x2d, w2d = pl.pallas_call(
        _critic_kernel,
        grid=(grid,),
        in_specs=[
            row_spec(obs.shape[1]),
            row_spec(mypose_goalpose.shape[1]),
            row_spec(NACT),
            row_spec(NACT),
            full(W_fc), full(b_fc),
            full(W_attn_in), full(b_attn_in),
            full(W_attn_w), full(b_attn_w),
            full(W_val), full(b_val),
        ],
        out_specs=[row_spec(NA), row_spec(NA)],
        out_shape=[
            jax.ShapeDtypeStruct((n, NA), jnp.float32),
            jax.ShapeDtypeStruct((n, NA), jnp.float32),
        ],
    )(obs, mypose_goalpose, policies, actions,
      W_fc, b_fc, W_attn_in, b_attn_in, W_attn_w, b_attn_w, W_val, b_val)

    return x2d[:, :, None], w2d[:, :, None]
